# baseline (speedup 1.0000x reference)
import functools

import jax
import jax.numpy as jnp
from jax import lax
from jax.experimental import pallas as pl
from jax.experimental.pallas import tpu as pltpu
from jax.experimental.pallas import tpu_sc as plsc

_L = 16


def _make_emb_kernel(B, V, D, NC, NS):
    NW = NC * NS
    b_per_w = B // NW
    n_grp = b_per_w // _L

    mesh = plsc.VectorSubcoreMesh(core_axis_name="c", subcore_axis_name="s")

    @functools.partial(
        pl.kernel,
        out_type=jax.ShapeDtypeStruct((4, 8, B), jnp.float32),
        mesh=mesh,
        scratch_types=[
            pltpu.VMEM((b_per_w,), jnp.int32),
            pltpu.VMEM((D, b_per_w), jnp.float32),
            pltpu.SemaphoreType.DMA,
            pltpu.SemaphoreType.DMA,
        ],
        compiler_params=pltpu.CompilerParams(
            needs_layout_passes=False, use_tc_tiling_on_sc=False
        ),
    )
    def emb_kernel(idx_hbm, tab1, out3, idx_v, gat_v, sem, osem):
        wid = lax.axis_index("s") * NC + lax.axis_index("c")
        pltpu.sync_copy(idx_hbm.at[wid], idx_v)

        def zero(g, _):
            z = jnp.zeros((_L,), jnp.float32)
            for d in range(D):
                gat_v[d, pl.ds(g * _L, _L)] = z
            return ()

        lax.fori_loop(0, n_grp, zero, (), unroll=False)

        def fire(g, _):
            iv = idx_v[pl.ds(g * _L, _L)]
            for d in range(D):
                off = iv + jnp.int32(d * V)
                pltpu.async_copy(
                    tab1.at[off],
                    gat_v.at[d, pl.ds(g * _L, _L)],
                    sem,
                )
            return ()

        lax.fori_loop(0, n_grp, fire, (), unroll=False)
        for t in range(4):
            pltpu.make_async_copy(
                out3.at[t, :, pl.ds(0, b_per_w)],
                gat_v.at[pl.ds(t * 8, 8)],
                sem,
            ).wait()
        writes = []
        for d in range(D):
            writes.append(
                pltpu.async_copy(
                    gat_v.at[d],
                    out3.at[d // 8, d % 8, pl.ds(wid * b_per_w, b_per_w)],
                    osem,
                )
            )
        for w in writes:
            w.wait()

    return emb_kernel


def kernel(cls, cls_emb):
    (B,) = cls.shape
    V, D = cls_emb.shape
    info = plsc.get_sparse_core_info()
    NC, NS = info.num_cores, info.num_subcores
    NW = NC * NS
    idx = cls.astype(jnp.int32).reshape(NW, B // NW)
    tab1 = cls_emb.T.reshape(-1)
    out3 = _make_emb_kernel(B, V, D, NC, NS)(idx, tab1)
    return out3.reshape(D, B).T
